# baseline (speedup 1.0000x reference)
import functools

import jax
import jax.numpy as jnp
from jax import lax
from jax.experimental import pallas as pl
from jax.experimental.pallas import tpu as pltpu
from jax.experimental.pallas import tpu_sc as plsc

_NW = 32
_NC = 2


def _build_pack(v, d):
    nblk = v // 128
    nmain = (nblk // _NW) * _NW
    slots = nmain // _NW
    nextra = nblk - nmain
    ntail = v - nblk * 128
    mesh = plsc.VectorSubcoreMesh(core_axis_name="c", subcore_axis_name="s")

    @functools.partial(
        pl.kernel,
        mesh=mesh,
        out_type=jax.ShapeDtypeStruct((v * d,), jnp.float32),
        scratch_types=[
            pltpu.VMEM((d, 128), jnp.float32),
            pltpu.VMEM((d, 128), jnp.float32),
            pltpu.VMEM((128 * d,), jnp.float32),
            pltpu.VMEM((128 * d,), jnp.float32),
            pltpu.VMEM((d, ntail), jnp.float32),
            pltpu.SemaphoreType.DMA,
            pltpu.SemaphoreType.DMA,
            pltpu.SemaphoreType.DMA,
            pltpu.SemaphoreType.DMA,
        ],
        compiler_params=pltpu.CompilerParams(
            use_tc_tiling_on_sc=True, needs_layout_passes=False),
    )
    def pack_kernel(tt_hbm, tail_hbm, out_hbm,
                    slab0, slab1, trows0, trows1, tailv, sr0, sr1, sw0, sw1):
        wid = lax.axis_index("s") * _NC + lax.axis_index("c")
        lane = lax.iota(jnp.int32, 16)

        def col_of(slot):
            return pl.multiple_of((slot * _NW + wid) * 128, 128)

        def transpose_block(slab, trows):
            def jbody(j, _):
                for m in range(8):
                    vals = slab[j, pl.ds(m * 16, 16)]
                    pos = (m * 16 + lane) * d + j
                    plsc.store_scatter(trows, [pos], vals)
                return ()
            lax.fori_loop(0, d, jbody, (), unroll=False)

        def rd(slot, slab, sem):
            return pltpu.async_copy(tt_hbm.at[:, pl.ds(col_of(slot), 128)],
                                    slab, sem)

        def wr(slot, trows, sem):
            off = pl.multiple_of(col_of(slot) * d, 8)
            return pltpu.async_copy(trows, out_hbm.at[pl.ds(off, 128 * d)], sem)

        rd(0, slab0, sr0)
        def body(i, _):
            slot_a = 2 * i
            slot_b = 2 * i + 1
            pltpu.make_async_copy(
                tt_hbm.at[:, pl.ds(col_of(slot_a), 128)], slab0, sr0).wait()
            rd(slot_b, slab1, sr1)

            @pl.when(i > 0)
            def _():
                pltpu.make_async_copy(
                    trows0, out_hbm.at[pl.ds(0, 128 * d)], sw0).wait()
            transpose_block(slab0, trows0)
            wr(slot_a, trows0, sw0)

            @pl.when(i < slots // 2 - 1)
            def _():
                rd(slot_a + 2, slab0, sr0)

            @pl.when(i > 0)
            def _():
                pltpu.make_async_copy(
                    trows1, out_hbm.at[pl.ds(0, 128 * d)], sw1).wait()
            pltpu.make_async_copy(
                tt_hbm.at[:, pl.ds(col_of(slot_b), 128)], slab1, sr1).wait()
            transpose_block(slab1, trows1)
            wr(slot_b, trows1, sw1)
            return ()

        lax.fori_loop(0, slots // 2, body, (), unroll=False)
        pltpu.make_async_copy(trows0, out_hbm.at[pl.ds(0, 128 * d)], sw0).wait()
        pltpu.make_async_copy(trows1, out_hbm.at[pl.ds(0, 128 * d)], sw1).wait()

        @pl.when(wid < nextra)
        def _():
            col0 = pl.multiple_of((nmain + wid) * 128, 128)
            pltpu.sync_copy(tt_hbm.at[:, pl.ds(col0, 128)], slab0)
            transpose_block(slab0, trows0)
            pltpu.sync_copy(trows0,
                            out_hbm.at[pl.ds(pl.multiple_of(col0 * d, 8),
                                             128 * d)])

        @pl.when(wid == _NW - 1)
        def _():
            pltpu.sync_copy(tail_hbm, tailv)
            def jbody(j, _):
                for m in range(ntail // 16):
                    vals = tailv[j, pl.ds(m * 16, 16)]
                    pos = (m * 16 + lane) * d + j
                    plsc.store_scatter(trows0, [pos], vals)
                return ()
            lax.fori_loop(0, d, jbody, (), unroll=False)
            pltpu.sync_copy(trows0.at[pl.ds(0, ntail * d)],
                            out_hbm.at[pl.ds(nblk * 128 * d, ntail * d)])

    return pack_kernel


def _build_gather(n, d, chunk):
    n_per_w = n // _NW
    n_chunks = n_per_w // chunk
    mesh = plsc.VectorSubcoreMesh(core_axis_name="c", subcore_axis_name="s")

    @functools.partial(
        pl.kernel,
        mesh=mesh,
        out_type=jax.ShapeDtypeStruct((n, d), jnp.float32),
        scratch_types=[
            pltpu.VMEM((chunk,), jnp.int32),
            pltpu.VMEM((chunk,), jnp.int32),
            pltpu.VMEM((chunk, d), jnp.float32),
            pltpu.VMEM((chunk, d), jnp.float32),
            pltpu.SemaphoreType.DMA,
            pltpu.SemaphoreType.DMA,
            pltpu.SemaphoreType.DMA,
            pltpu.SemaphoreType.DMA,
            pltpu.SemaphoreType.DMA,
            pltpu.SemaphoreType.DMA,
        ],
        compiler_params=pltpu.CompilerParams(use_tc_tiling_on_sc=False),
    )
    def gather_kernel(table_hbm, idx_hbm, out_hbm,
                      idx0, idx1, rows0, rows1, si0, si1, sg0, sg1, ss0, ss1):
        wid = lax.axis_index("s") * _NC + lax.axis_index("c")
        base = wid * n_per_w
        idx_b = [idx0, idx1]
        rows_b = [rows0, rows1]
        si = [si0, si1]
        sg = [sg0, sg1]
        ss = [ss0, ss1]

        def off(c):
            return base + c * chunk

        h_idx = [
            pltpu.async_copy(idx_hbm.at[pl.ds(off(0), chunk)], idx_b[0], si[0]),
            pltpu.async_copy(idx_hbm.at[pl.ds(off(1), chunk)], idx_b[1], si[1]),
        ]
        h_s = [None, None]
        for c in range(n_chunks):
            p = c % 2
            if c >= 2:
                h_s[p].wait()
            h_idx[p].wait()
            pltpu.async_copy(table_hbm.at[idx_b[p]], rows_b[p], sg[p]).wait()
            if c + 2 < n_chunks:
                h_idx[p] = pltpu.async_copy(
                    idx_hbm.at[pl.ds(off(c + 2), chunk)], idx_b[p], si[p])
            h_s[p] = pltpu.async_copy(
                rows_b[p], out_hbm.at[pl.ds(off(c), chunk)], ss[p])
        h_s[0].wait()
        h_s[1].wait()

    return gather_kernel


def kernel(x, table):
    b, l = x.shape
    v, d = table.shape
    n = b * l
    tt = table.T
    nfull = (v // 128) * 128
    tail_t = lax.slice(tt, (0, nfull), (d, v))
    tlin = _build_pack(v, d)(tt, tail_t)
    idx = x.reshape(n).astype(jnp.int32)
    out = _build_gather(n, d, 512)(tlin.reshape(v, d), idx)
    return out.reshape(b, l, d)


# speedup vs baseline: 1.6777x; 1.6777x over previous
import functools

import jax
import jax.numpy as jnp
from jax import lax
from jax.experimental import pallas as pl
from jax.experimental.pallas import tpu as pltpu
from jax.experimental.pallas import tpu_sc as plsc

_NW = 32
_NC = 2


def _build_pack(v, d):
    nblk = v // 128
    nmain = (nblk // _NW) * _NW
    slots = nmain // _NW
    nextra = nblk - nmain
    ntail = v - nblk * 128
    mesh = plsc.VectorSubcoreMesh(core_axis_name="c", subcore_axis_name="s")

    @functools.partial(
        pl.kernel,
        mesh=mesh,
        out_type=jax.ShapeDtypeStruct((v * d,), jnp.float32),
        scratch_types=[
            pltpu.VMEM((d, 128), jnp.float32),
            pltpu.VMEM((d, 128), jnp.float32),
            pltpu.VMEM((128 * d,), jnp.float32),
            pltpu.VMEM((128 * d,), jnp.float32),
            pltpu.VMEM((d, ntail), jnp.float32),
            pltpu.SemaphoreType.DMA,
            pltpu.SemaphoreType.DMA,
            pltpu.SemaphoreType.DMA,
            pltpu.SemaphoreType.DMA,
        ],
        compiler_params=pltpu.CompilerParams(
            use_tc_tiling_on_sc=True, needs_layout_passes=False),
    )
    def pack_kernel(tt_hbm, tail_hbm, out_hbm,
                    slab0, slab1, trows0, trows1, tailv, sr0, sr1, sw0, sw1):
        wid = lax.axis_index("s") * _NC + lax.axis_index("c")
        lane = lax.iota(jnp.int32, 16)

        def col_of(slot):
            return pl.multiple_of((slot * _NW + wid) * 128, 128)

        rot = [(lane + dd) & 15 for dd in range(16)]
        pos_pat = [lane * d + ((lane + dd) & 15) for dd in range(16)]

        def transpose_block(slab, trows):
            for jt in range(d // 16):
                j0 = jt * 16

                def cbody(ct, _):
                    c0 = ct * 16
                    cvec = c0 + lane
                    base = c0 * d + j0
                    for dd in range(16):
                        vals = plsc.load_gather(slab, [j0 + rot[dd], cvec])
                        plsc.store_scatter(trows, [base + pos_pat[dd]], vals)
                    return ()

                lax.fori_loop(0, 8, cbody, (), unroll=2)

        def rd(slot, slab, sem):
            return pltpu.async_copy(tt_hbm.at[:, pl.ds(col_of(slot), 128)],
                                    slab, sem)

        def wr(slot, trows, sem):
            off = pl.multiple_of(col_of(slot) * d, 8)
            return pltpu.async_copy(trows, out_hbm.at[pl.ds(off, 128 * d)], sem)

        rd(0, slab0, sr0)
        def body(i, _):
            slot_a = 2 * i
            slot_b = 2 * i + 1
            pltpu.make_async_copy(
                tt_hbm.at[:, pl.ds(col_of(slot_a), 128)], slab0, sr0).wait()
            rd(slot_b, slab1, sr1)

            @pl.when(i > 0)
            def _():
                pltpu.make_async_copy(
                    trows0, out_hbm.at[pl.ds(0, 128 * d)], sw0).wait()
            transpose_block(slab0, trows0)
            wr(slot_a, trows0, sw0)

            @pl.when(i < slots // 2 - 1)
            def _():
                rd(slot_a + 2, slab0, sr0)

            @pl.when(i > 0)
            def _():
                pltpu.make_async_copy(
                    trows1, out_hbm.at[pl.ds(0, 128 * d)], sw1).wait()
            pltpu.make_async_copy(
                tt_hbm.at[:, pl.ds(col_of(slot_b), 128)], slab1, sr1).wait()
            transpose_block(slab1, trows1)
            wr(slot_b, trows1, sw1)
            return ()

        lax.fori_loop(0, slots // 2, body, (), unroll=False)
        pltpu.make_async_copy(trows0, out_hbm.at[pl.ds(0, 128 * d)], sw0).wait()
        pltpu.make_async_copy(trows1, out_hbm.at[pl.ds(0, 128 * d)], sw1).wait()

        @pl.when(wid < nextra)
        def _():
            col0 = pl.multiple_of((nmain + wid) * 128, 128)
            pltpu.sync_copy(tt_hbm.at[:, pl.ds(col0, 128)], slab0)
            transpose_block(slab0, trows0)
            pltpu.sync_copy(trows0,
                            out_hbm.at[pl.ds(pl.multiple_of(col0 * d, 8),
                                             128 * d)])

        @pl.when(wid == _NW - 1)
        def _():
            pltpu.sync_copy(tail_hbm, tailv)
            def jbody(j, _):
                for m in range(ntail // 16):
                    vals = tailv[j, pl.ds(m * 16, 16)]
                    pos = (m * 16 + lane) * d + j
                    plsc.store_scatter(trows0, [pos], vals)
                return ()
            lax.fori_loop(0, d, jbody, (), unroll=False)
            pltpu.sync_copy(trows0.at[pl.ds(0, ntail * d)],
                            out_hbm.at[pl.ds(nblk * 128 * d, ntail * d)])

    return pack_kernel


def _build_gather(n, d, chunk):
    n_per_w = n // _NW
    n_chunks = n_per_w // chunk
    mesh = plsc.VectorSubcoreMesh(core_axis_name="c", subcore_axis_name="s")

    @functools.partial(
        pl.kernel,
        mesh=mesh,
        out_type=jax.ShapeDtypeStruct((n, d), jnp.float32),
        scratch_types=[
            pltpu.VMEM((chunk,), jnp.int32),
            pltpu.VMEM((chunk,), jnp.int32),
            pltpu.VMEM((chunk, d), jnp.float32),
            pltpu.VMEM((chunk, d), jnp.float32),
            pltpu.SemaphoreType.DMA,
            pltpu.SemaphoreType.DMA,
            pltpu.SemaphoreType.DMA,
            pltpu.SemaphoreType.DMA,
            pltpu.SemaphoreType.DMA,
            pltpu.SemaphoreType.DMA,
        ],
        compiler_params=pltpu.CompilerParams(use_tc_tiling_on_sc=False),
    )
    def gather_kernel(table_hbm, idx_hbm, out_hbm,
                      idx0, idx1, rows0, rows1, si0, si1, sg0, sg1, ss0, ss1):
        wid = lax.axis_index("s") * _NC + lax.axis_index("c")
        base = wid * n_per_w
        idx_b = [idx0, idx1]
        rows_b = [rows0, rows1]
        si = [si0, si1]
        sg = [sg0, sg1]
        ss = [ss0, ss1]

        def off(c):
            return base + c * chunk

        h_idx = [
            pltpu.async_copy(idx_hbm.at[pl.ds(off(0), chunk)], idx_b[0], si[0]),
            pltpu.async_copy(idx_hbm.at[pl.ds(off(1), chunk)], idx_b[1], si[1]),
        ]
        h_s = [None, None]
        for c in range(n_chunks):
            p = c % 2
            if c >= 2:
                h_s[p].wait()
            h_idx[p].wait()
            pltpu.async_copy(table_hbm.at[idx_b[p]], rows_b[p], sg[p]).wait()
            if c + 2 < n_chunks:
                h_idx[p] = pltpu.async_copy(
                    idx_hbm.at[pl.ds(off(c + 2), chunk)], idx_b[p], si[p])
            h_s[p] = pltpu.async_copy(
                rows_b[p], out_hbm.at[pl.ds(off(c), chunk)], ss[p])
        h_s[0].wait()
        h_s[1].wait()

    return gather_kernel


def kernel(x, table):
    b, l = x.shape
    v, d = table.shape
    n = b * l
    tt = table.T
    nfull = (v // 128) * 128
    tail_t = lax.slice(tt, (0, nfull), (d, v))
    tlin = _build_pack(v, d)(tt, tail_t)
    idx = x.reshape(n).astype(jnp.int32)
    out = _build_gather(n, d, 512)(tlin.reshape(v, d), idx)
    return out.reshape(b, l, d)


# speedup vs baseline: 2.7519x; 1.6403x over previous
import functools

import jax
import jax.numpy as jnp
from jax import lax
from jax.experimental import pallas as pl
from jax.experimental.pallas import tpu as pltpu
from jax.experimental.pallas import tpu_sc as plsc

_NW = 32
_NC = 2


def _build_pack(v, d):
    nblk = v // 128
    nmain = (nblk // _NW) * _NW
    slots = nmain // _NW
    nextra = nblk - nmain
    ntail = v - nblk * 128
    mesh = plsc.VectorSubcoreMesh(core_axis_name="c", subcore_axis_name="s")

    @functools.partial(
        pl.kernel,
        mesh=mesh,
        out_type=jax.ShapeDtypeStruct((v * d,), jnp.float32),
        scratch_types=[
            pltpu.VMEM((d, 128), jnp.float32),
            pltpu.VMEM((d, 128), jnp.float32),
            pltpu.VMEM((128 * d,), jnp.float32),
            pltpu.VMEM((128 * d,), jnp.float32),
            pltpu.VMEM((d, ntail), jnp.float32),
            pltpu.SemaphoreType.DMA,
            pltpu.SemaphoreType.DMA,
            pltpu.SemaphoreType.DMA,
            pltpu.SemaphoreType.DMA,
        ],
        compiler_params=pltpu.CompilerParams(
            use_tc_tiling_on_sc=True, needs_layout_passes=False),
    )
    def pack_kernel(tt_hbm, tail_hbm, out_hbm,
                    slab0, slab1, trows0, trows1, tailv, sr0, sr1, sw0, sw1):
        wid = lax.axis_index("s") * _NC + lax.axis_index("c")
        lane = lax.iota(jnp.int32, 16)

        def col_of(slot):
            return pl.multiple_of((slot * _NW + wid) * 128, 128)

        rot = [(lane + dd) & 15 for dd in range(16)]
        pos_pat = [lane * d + ((lane + dd) & 15) for dd in range(16)]

        def transpose_block(slab, trows):
            for jt in range(d // 16):
                j0 = jt * 16

                def cbody(ct, _):
                    c0 = ct * 16
                    cvec = c0 + lane
                    base = c0 * d + j0
                    for g in range(0, 16, 8):
                        vals = [plsc.load_gather(slab, [j0 + rot[dd], cvec])
                                for dd in range(g, g + 8)]
                        for k, dd in enumerate(range(g, g + 8)):
                            plsc.store_scatter(
                                trows, [base + pos_pat[dd]], vals[k])
                    return ()

                lax.fori_loop(0, 8, cbody, (), unroll=1)

        def rd(slot, slab, sem):
            return pltpu.async_copy(tt_hbm.at[:, pl.ds(col_of(slot), 128)],
                                    slab, sem)

        def wr(slot, trows, sem):
            off = pl.multiple_of(col_of(slot) * d, 8)
            return pltpu.async_copy(trows, out_hbm.at[pl.ds(off, 128 * d)], sem)

        rd(0, slab0, sr0)
        def body(i, _):
            slot_a = 2 * i
            slot_b = 2 * i + 1
            pltpu.make_async_copy(
                tt_hbm.at[:, pl.ds(col_of(slot_a), 128)], slab0, sr0).wait()
            rd(slot_b, slab1, sr1)

            @pl.when(i > 0)
            def _():
                pltpu.make_async_copy(
                    trows0, out_hbm.at[pl.ds(0, 128 * d)], sw0).wait()
            transpose_block(slab0, trows0)
            wr(slot_a, trows0, sw0)

            @pl.when(i < slots // 2 - 1)
            def _():
                rd(slot_a + 2, slab0, sr0)

            @pl.when(i > 0)
            def _():
                pltpu.make_async_copy(
                    trows1, out_hbm.at[pl.ds(0, 128 * d)], sw1).wait()
            pltpu.make_async_copy(
                tt_hbm.at[:, pl.ds(col_of(slot_b), 128)], slab1, sr1).wait()
            transpose_block(slab1, trows1)
            wr(slot_b, trows1, sw1)
            return ()

        lax.fori_loop(0, slots // 2, body, (), unroll=False)
        pltpu.make_async_copy(trows0, out_hbm.at[pl.ds(0, 128 * d)], sw0).wait()
        pltpu.make_async_copy(trows1, out_hbm.at[pl.ds(0, 128 * d)], sw1).wait()

        @pl.when(wid < nextra)
        def _():
            col0 = pl.multiple_of((nmain + wid) * 128, 128)
            pltpu.sync_copy(tt_hbm.at[:, pl.ds(col0, 128)], slab0)
            transpose_block(slab0, trows0)
            pltpu.sync_copy(trows0,
                            out_hbm.at[pl.ds(pl.multiple_of(col0 * d, 8),
                                             128 * d)])

        @pl.when(wid == _NW - 1)
        def _():
            pltpu.sync_copy(tail_hbm, tailv)
            def jbody(j, _):
                for m in range(ntail // 16):
                    vals = tailv[j, pl.ds(m * 16, 16)]
                    pos = (m * 16 + lane) * d + j
                    plsc.store_scatter(trows0, [pos], vals)
                return ()
            lax.fori_loop(0, d, jbody, (), unroll=False)
            pltpu.sync_copy(trows0.at[pl.ds(0, ntail * d)],
                            out_hbm.at[pl.ds(nblk * 128 * d, ntail * d)])

    return pack_kernel


def _build_gather(n, d, chunk):
    n_per_w = n // _NW
    n_chunks = n_per_w // chunk
    mesh = plsc.VectorSubcoreMesh(core_axis_name="c", subcore_axis_name="s")

    @functools.partial(
        pl.kernel,
        mesh=mesh,
        out_type=jax.ShapeDtypeStruct((n, d), jnp.float32),
        scratch_types=[
            pltpu.VMEM((chunk,), jnp.int32),
            pltpu.VMEM((chunk,), jnp.int32),
            pltpu.VMEM((chunk, d), jnp.float32),
            pltpu.VMEM((chunk, d), jnp.float32),
            pltpu.SemaphoreType.DMA,
            pltpu.SemaphoreType.DMA,
            pltpu.SemaphoreType.DMA,
            pltpu.SemaphoreType.DMA,
            pltpu.SemaphoreType.DMA,
            pltpu.SemaphoreType.DMA,
        ],
        compiler_params=pltpu.CompilerParams(use_tc_tiling_on_sc=False),
    )
    def gather_kernel(table_hbm, idx_hbm, out_hbm,
                      idx0, idx1, rows0, rows1, si0, si1, sg0, sg1, ss0, ss1):
        wid = lax.axis_index("s") * _NC + lax.axis_index("c")
        base = wid * n_per_w
        idx_b = [idx0, idx1]
        rows_b = [rows0, rows1]
        si = [si0, si1]
        sg = [sg0, sg1]
        ss = [ss0, ss1]

        def off(c):
            return base + c * chunk

        h_idx = [
            pltpu.async_copy(idx_hbm.at[pl.ds(off(0), chunk)], idx_b[0], si[0]),
            pltpu.async_copy(idx_hbm.at[pl.ds(off(1), chunk)], idx_b[1], si[1]),
        ]
        h_s = [None, None]
        for c in range(n_chunks):
            p = c % 2
            if c >= 2:
                h_s[p].wait()
            h_idx[p].wait()
            pltpu.async_copy(table_hbm.at[idx_b[p]], rows_b[p], sg[p]).wait()
            if c + 2 < n_chunks:
                h_idx[p] = pltpu.async_copy(
                    idx_hbm.at[pl.ds(off(c + 2), chunk)], idx_b[p], si[p])
            h_s[p] = pltpu.async_copy(
                rows_b[p], out_hbm.at[pl.ds(off(c), chunk)], ss[p])
        h_s[0].wait()
        h_s[1].wait()

    return gather_kernel


def kernel(x, table):
    b, l = x.shape
    v, d = table.shape
    n = b * l
    tt = table.T
    nfull = (v // 128) * 128
    tail_t = lax.slice(tt, (0, nfull), (d, v))
    tlin = _build_pack(v, d)(tt, tail_t)
    idx = x.reshape(n).astype(jnp.int32)
    out = _build_gather(n, d, 512)(tlin.reshape(v, d), idx)
    return out.reshape(b, l, d)


# speedup vs baseline: 3.0193x; 1.0972x over previous
import functools

import jax
import jax.numpy as jnp
from jax import lax
from jax.experimental import pallas as pl
from jax.experimental.pallas import tpu as pltpu
from jax.experimental.pallas import tpu_sc as plsc

_NW = 32
_NC = 2


def _build_pack(v, d):
    nblk = v // 128
    nmain = (nblk // _NW) * _NW
    slots = nmain // _NW
    nextra = nblk - nmain
    ntail = v - nblk * 128
    mesh = plsc.VectorSubcoreMesh(core_axis_name="c", subcore_axis_name="s")

    @functools.partial(
        pl.kernel,
        mesh=mesh,
        out_type=jax.ShapeDtypeStruct((v * d,), jnp.float32),
        scratch_types=[
            pltpu.VMEM((d, 128), jnp.float32),
            pltpu.VMEM((d, 128), jnp.float32),
            pltpu.VMEM((128 * d,), jnp.float32),
            pltpu.VMEM((128 * d,), jnp.float32),
            pltpu.VMEM((d, ntail), jnp.float32),
            pltpu.SemaphoreType.DMA,
            pltpu.SemaphoreType.DMA,
            pltpu.SemaphoreType.DMA,
            pltpu.SemaphoreType.DMA,
        ],
        compiler_params=pltpu.CompilerParams(
            use_tc_tiling_on_sc=True, needs_layout_passes=False),
    )
    def pack_kernel(tt_hbm, tail_hbm, out_hbm,
                    slab0, slab1, trows0, trows1, tailv, sr0, sr1, sw0, sw1):
        wid = lax.axis_index("s") * _NC + lax.axis_index("c")
        lane = lax.iota(jnp.int32, 16)

        def col_of(slot):
            return pl.multiple_of((slot * _NW + wid) * 128, 128)

        rot = [(lane + dd) & 15 for dd in range(16)]
        pos_pat = [lane * d + ((lane + dd) & 15) for dd in range(16)]

        def transpose_block(slab, trows):
            for jt in range(d // 16):
                j0 = jt * 16

                def cbody(ct, _):
                    c0 = ct * 16
                    cvec = c0 + lane
                    base = c0 * d + j0
                    for g in range(0, 16, 8):
                        vals = [plsc.load_gather(slab, [j0 + rot[dd], cvec])
                                for dd in range(g, g + 8)]
                        for k, dd in enumerate(range(g, g + 8)):
                            plsc.store_scatter(
                                trows, [base + pos_pat[dd]], vals[k])
                    return ()

                lax.fori_loop(0, 8, cbody, (), unroll=1)

        def rd(slot, slab, sem):
            return pltpu.async_copy(tt_hbm.at[:, pl.ds(col_of(slot), 128)],
                                    slab, sem)

        def wr(slot, trows, sem):
            off = pl.multiple_of(col_of(slot) * d, 8)
            return pltpu.async_copy(trows, out_hbm.at[pl.ds(off, 128 * d)], sem)

        rd(0, slab0, sr0)
        def body(i, _):
            slot_a = 2 * i
            slot_b = 2 * i + 1
            pltpu.make_async_copy(
                tt_hbm.at[:, pl.ds(col_of(slot_a), 128)], slab0, sr0).wait()
            rd(slot_b, slab1, sr1)

            @pl.when(i > 0)
            def _():
                pltpu.make_async_copy(
                    trows0, out_hbm.at[pl.ds(0, 128 * d)], sw0).wait()
            transpose_block(slab0, trows0)
            wr(slot_a, trows0, sw0)

            @pl.when(i < slots // 2 - 1)
            def _():
                rd(slot_a + 2, slab0, sr0)

            @pl.when(i > 0)
            def _():
                pltpu.make_async_copy(
                    trows1, out_hbm.at[pl.ds(0, 128 * d)], sw1).wait()
            pltpu.make_async_copy(
                tt_hbm.at[:, pl.ds(col_of(slot_b), 128)], slab1, sr1).wait()
            transpose_block(slab1, trows1)
            wr(slot_b, trows1, sw1)
            return ()

        lax.fori_loop(0, slots // 2, body, (), unroll=False)
        pltpu.make_async_copy(trows0, out_hbm.at[pl.ds(0, 128 * d)], sw0).wait()
        pltpu.make_async_copy(trows1, out_hbm.at[pl.ds(0, 128 * d)], sw1).wait()

        @pl.when(wid < nextra)
        def _():
            col0 = pl.multiple_of((nmain + wid) * 128, 128)
            pltpu.sync_copy(tt_hbm.at[:, pl.ds(col0, 128)], slab0)
            transpose_block(slab0, trows0)
            pltpu.sync_copy(trows0,
                            out_hbm.at[pl.ds(pl.multiple_of(col0 * d, 8),
                                             128 * d)])

        @pl.when(wid == _NW - 1)
        def _():
            pltpu.sync_copy(tail_hbm, tailv)
            def jbody(j, _):
                for m in range(ntail // 16):
                    vals = tailv[j, pl.ds(m * 16, 16)]
                    pos = (m * 16 + lane) * d + j
                    plsc.store_scatter(trows0, [pos], vals)
                return ()
            lax.fori_loop(0, d, jbody, (), unroll=False)
            pltpu.sync_copy(trows0.at[pl.ds(0, ntail * d)],
                            out_hbm.at[pl.ds(nblk * 128 * d, ntail * d)])

    return pack_kernel


def _build_gather(n, d, chunk):
    n_per_w = n // _NW
    n_chunks = n_per_w // chunk
    mesh = plsc.VectorSubcoreMesh(core_axis_name="c", subcore_axis_name="s")

    @functools.partial(
        pl.kernel,
        mesh=mesh,
        out_type=jax.ShapeDtypeStruct((n, d), jnp.float32),
        scratch_types=[
            pltpu.VMEM((chunk,), jnp.int32),
            pltpu.VMEM((chunk,), jnp.int32),
            pltpu.VMEM((chunk, d), jnp.float32),
            pltpu.VMEM((chunk, d), jnp.float32),
            pltpu.SemaphoreType.DMA,
            pltpu.SemaphoreType.DMA,
            pltpu.SemaphoreType.DMA,
            pltpu.SemaphoreType.DMA,
            pltpu.SemaphoreType.DMA,
            pltpu.SemaphoreType.DMA,
        ],
        compiler_params=pltpu.CompilerParams(use_tc_tiling_on_sc=False),
    )
    def gather_kernel(table_hbm, idx_hbm, out_hbm,
                      idx0, idx1, rows0, rows1, si0, si1, sg0, sg1, ss0, ss1):
        wid = lax.axis_index("s") * _NC + lax.axis_index("c")
        base = wid * n_per_w
        idx_b = [idx0, idx1]
        rows_b = [rows0, rows1]
        si = [si0, si1]
        sg = [sg0, sg1]
        ss = [ss0, ss1]

        def off(c):
            return base + c * chunk

        h_idx = [
            pltpu.async_copy(idx_hbm.at[pl.ds(off(0), chunk)], idx_b[0], si[0]),
            pltpu.async_copy(idx_hbm.at[pl.ds(off(1), chunk)], idx_b[1], si[1]),
        ]
        h_s = [None, None]
        for c in range(n_chunks):
            p = c % 2
            if c >= 2:
                h_s[p].wait()
            h_idx[p].wait()
            pltpu.async_copy(table_hbm.at[idx_b[p]], rows_b[p], sg[p]).wait()
            if c + 2 < n_chunks:
                h_idx[p] = pltpu.async_copy(
                    idx_hbm.at[pl.ds(off(c + 2), chunk)], idx_b[p], si[p])
            h_s[p] = pltpu.async_copy(
                rows_b[p], out_hbm.at[pl.ds(off(c), chunk)], ss[p])
        h_s[0].wait()
        h_s[1].wait()

    return gather_kernel


def _build_gather_t(b, l, d, half):
    b_per_w = b // _NW
    nh = b_per_w // half
    assert nh == 2
    mesh = plsc.VectorSubcoreMesh(core_axis_name="c", subcore_axis_name="s")

    @functools.partial(
        pl.kernel,
        mesh=mesh,
        out_type=jax.ShapeDtypeStruct((l, d, b), jnp.float32),
        scratch_types=[
            pltpu.VMEM((half,), jnp.int32),
            pltpu.VMEM((half,), jnp.int32),
            pltpu.VMEM((half, d), jnp.float32),
            pltpu.VMEM((half, d), jnp.float32),
            pltpu.VMEM((d, half), jnp.float32),
            pltpu.VMEM((d, half), jnp.float32),
            pltpu.SemaphoreType.DMA,
            pltpu.SemaphoreType.DMA,
            pltpu.SemaphoreType.DMA,
            pltpu.SemaphoreType.DMA,
            pltpu.SemaphoreType.DMA,
            pltpu.SemaphoreType.DMA,
        ],
        compiler_params=pltpu.CompilerParams(
            use_tc_tiling_on_sc=False, needs_layout_passes=False),
    )
    def gather_t_kernel(table_hbm, xt_hbm, out_hbm,
                        idx0, idx1, rows0, rows1, tc0, tc1,
                        si0, si1, sg0, sg1, sw0, sw1):
        wid = lax.axis_index("s") * _NC + lax.axis_index("c")
        b0 = wid * b_per_w
        lane = lax.iota(jnp.int32, 16)
        rot = [(lane + dd) & 15 for dd in range(16)]

        def transpose_chunk(rows, tcols):
            def rbody(rt, _):
                rvec = rt * 16 + lane
                for jt in range(d // 16):
                    for g in range(0, 16, 8):
                        jv = [jt * 16 + rot[dd] for dd in range(g, g + 8)]
                        vals = [plsc.load_gather(rows, [rvec, jv[k]])
                                for k in range(8)]
                        for k in range(8):
                            plsc.store_scatter(tcols, [jv[k], rvec], vals[k])
                return ()
            lax.fori_loop(0, half // 16, rbody, (), unroll=1)

        def idx_src(li, h):
            return xt_hbm.at[li, pl.ds(b0 + h * half, half)]

        def out_dst(li, h):
            return out_hbm.at[li, :, pl.ds(b0 + h * half, half)]

        pltpu.async_copy(idx_src(0, 0), idx0, si0)
        pltpu.async_copy(idx_src(0, 1), idx1, si1)

        def body(i, _):
            pltpu.make_async_copy(idx_src(i, 0), idx0, si0).wait()
            pltpu.async_copy(table_hbm.at[idx0], rows0, sg0).wait()

            @pl.when(i < l - 1)
            def _():
                pltpu.async_copy(idx_src(i + 1, 0), idx0, si0)

            @pl.when(i > 0)
            def _():
                pltpu.make_async_copy(tc0, out_dst(i, 0), sw0).wait()
            transpose_chunk(rows0, tc0)
            pltpu.async_copy(tc0, out_dst(i, 0), sw0)

            pltpu.make_async_copy(idx_src(i, 1), idx1, si1).wait()
            pltpu.async_copy(table_hbm.at[idx1], rows1, sg1).wait()

            @pl.when(i < l - 1)
            def _():
                pltpu.async_copy(idx_src(i + 1, 1), idx1, si1)

            @pl.when(i > 0)
            def _():
                pltpu.make_async_copy(tc1, out_dst(i, 1), sw1).wait()
            transpose_chunk(rows1, tc1)
            pltpu.async_copy(tc1, out_dst(i, 1), sw1)
            return ()

        lax.fori_loop(0, l, body, (), unroll=False)
        pltpu.make_async_copy(tc0, out_dst(l - 1, 0), sw0).wait()
        pltpu.make_async_copy(tc1, out_dst(l - 1, 1), sw1).wait()

    return gather_t_kernel


def kernel(x, table):
    b, l = x.shape
    v, d = table.shape
    tt = table.T
    nfull = (v // 128) * 128
    tail_t = lax.slice(tt, (0, nfull), (d, v))
    tlin = _build_pack(v, d)(tt, tail_t)
    xt = x.T.astype(jnp.int32)
    out_t = _build_gather_t(b, l, d, 256)(tlin.reshape(v, d), xt)
    return out_t.transpose(2, 0, 1)


# speedup vs baseline: 3.1490x; 1.0430x over previous
import functools

import jax
import jax.numpy as jnp
from jax import lax
from jax.experimental import pallas as pl
from jax.experimental.pallas import tpu as pltpu
from jax.experimental.pallas import tpu_sc as plsc

_NW = 32
_NC = 2


def _build_pack(v, d):
    nblk = v // 128
    nmain = (nblk // _NW) * _NW
    slots = nmain // _NW
    nextra = nblk - nmain
    ntail = v - nblk * 128
    mesh = plsc.VectorSubcoreMesh(core_axis_name="c", subcore_axis_name="s")

    @functools.partial(
        pl.kernel,
        mesh=mesh,
        out_type=jax.ShapeDtypeStruct((v * d,), jnp.float32),
        scratch_types=[
            pltpu.VMEM((d, 128), jnp.float32),
            pltpu.VMEM((d, 128), jnp.float32),
            pltpu.VMEM((128 * d,), jnp.float32),
            pltpu.VMEM((128 * d,), jnp.float32),
            pltpu.VMEM((d, ntail), jnp.float32),
            pltpu.SemaphoreType.DMA,
            pltpu.SemaphoreType.DMA,
            pltpu.SemaphoreType.DMA,
            pltpu.SemaphoreType.DMA,
        ],
        compiler_params=pltpu.CompilerParams(
            use_tc_tiling_on_sc=True, needs_layout_passes=False),
    )
    def pack_kernel(tt_hbm, tail_hbm, out_hbm,
                    slab0, slab1, trows0, trows1, tailv, sr0, sr1, sw0, sw1):
        wid = lax.axis_index("s") * _NC + lax.axis_index("c")
        lane = lax.iota(jnp.int32, 16)

        def col_of(slot):
            return pl.multiple_of((slot * _NW + wid) * 128, 128)

        rot = [(lane + dd) & 15 for dd in range(16)]
        pos_pat = [lane * d + ((lane + dd) & 15) for dd in range(16)]

        def transpose_block(slab, trows):
            for jt in range(d // 16):
                j0 = jt * 16

                def cbody(ct, _):
                    c0 = ct * 16
                    cvec = c0 + lane
                    base = c0 * d + j0
                    for g in range(0, 16, 8):
                        vals = [plsc.load_gather(slab, [j0 + rot[dd], cvec])
                                for dd in range(g, g + 8)]
                        for k, dd in enumerate(range(g, g + 8)):
                            plsc.store_scatter(
                                trows, [base + pos_pat[dd]], vals[k])
                    return ()

                lax.fori_loop(0, 8, cbody, (), unroll=1)

        def rd(slot, slab, sem):
            return pltpu.async_copy(tt_hbm.at[:, pl.ds(col_of(slot), 128)],
                                    slab, sem)

        def wr(slot, trows, sem):
            off = pl.multiple_of(col_of(slot) * d, 8)
            return pltpu.async_copy(trows, out_hbm.at[pl.ds(off, 128 * d)], sem)

        rd(0, slab0, sr0)
        def body(i, _):
            slot_a = 2 * i
            slot_b = 2 * i + 1
            pltpu.make_async_copy(
                tt_hbm.at[:, pl.ds(col_of(slot_a), 128)], slab0, sr0).wait()
            rd(slot_b, slab1, sr1)

            @pl.when(i > 0)
            def _():
                pltpu.make_async_copy(
                    trows0, out_hbm.at[pl.ds(0, 128 * d)], sw0).wait()
            transpose_block(slab0, trows0)
            wr(slot_a, trows0, sw0)

            @pl.when(i < slots // 2 - 1)
            def _():
                rd(slot_a + 2, slab0, sr0)

            @pl.when(i > 0)
            def _():
                pltpu.make_async_copy(
                    trows1, out_hbm.at[pl.ds(0, 128 * d)], sw1).wait()
            pltpu.make_async_copy(
                tt_hbm.at[:, pl.ds(col_of(slot_b), 128)], slab1, sr1).wait()
            transpose_block(slab1, trows1)
            wr(slot_b, trows1, sw1)
            return ()

        lax.fori_loop(0, slots // 2, body, (), unroll=False)
        pltpu.make_async_copy(trows0, out_hbm.at[pl.ds(0, 128 * d)], sw0).wait()
        pltpu.make_async_copy(trows1, out_hbm.at[pl.ds(0, 128 * d)], sw1).wait()

        @pl.when(wid < nextra)
        def _():
            col0 = pl.multiple_of((nmain + wid) * 128, 128)
            pltpu.sync_copy(tt_hbm.at[:, pl.ds(col0, 128)], slab0)
            transpose_block(slab0, trows0)
            pltpu.sync_copy(trows0,
                            out_hbm.at[pl.ds(pl.multiple_of(col0 * d, 8),
                                             128 * d)])

        @pl.when(wid == _NW - 1)
        def _():
            pltpu.sync_copy(tail_hbm, tailv)
            def jbody(j, _):
                for m in range(ntail // 16):
                    vals = tailv[j, pl.ds(m * 16, 16)]
                    pos = (m * 16 + lane) * d + j
                    plsc.store_scatter(trows0, [pos], vals)
                return ()
            lax.fori_loop(0, d, jbody, (), unroll=False)
            pltpu.sync_copy(trows0.at[pl.ds(0, ntail * d)],
                            out_hbm.at[pl.ds(nblk * 128 * d, ntail * d)])

    return pack_kernel


def _build_gather(n, d, chunk):
    n_per_w = n // _NW
    n_chunks = n_per_w // chunk
    mesh = plsc.VectorSubcoreMesh(core_axis_name="c", subcore_axis_name="s")

    @functools.partial(
        pl.kernel,
        mesh=mesh,
        out_type=jax.ShapeDtypeStruct((n, d), jnp.float32),
        scratch_types=[
            pltpu.VMEM((chunk,), jnp.int32),
            pltpu.VMEM((chunk,), jnp.int32),
            pltpu.VMEM((chunk, d), jnp.float32),
            pltpu.VMEM((chunk, d), jnp.float32),
            pltpu.SemaphoreType.DMA,
            pltpu.SemaphoreType.DMA,
            pltpu.SemaphoreType.DMA,
            pltpu.SemaphoreType.DMA,
            pltpu.SemaphoreType.DMA,
            pltpu.SemaphoreType.DMA,
        ],
        compiler_params=pltpu.CompilerParams(use_tc_tiling_on_sc=False),
    )
    def gather_kernel(table_hbm, idx_hbm, out_hbm,
                      idx0, idx1, rows0, rows1, si0, si1, sg0, sg1, ss0, ss1):
        wid = lax.axis_index("s") * _NC + lax.axis_index("c")
        base = wid * n_per_w
        idx_b = [idx0, idx1]
        rows_b = [rows0, rows1]
        si = [si0, si1]
        sg = [sg0, sg1]
        ss = [ss0, ss1]

        def off(c):
            return base + c * chunk

        h_idx = [
            pltpu.async_copy(idx_hbm.at[pl.ds(off(0), chunk)], idx_b[0], si[0]),
            pltpu.async_copy(idx_hbm.at[pl.ds(off(1), chunk)], idx_b[1], si[1]),
        ]
        h_s = [None, None]
        for c in range(n_chunks):
            p = c % 2
            if c >= 2:
                h_s[p].wait()
            h_idx[p].wait()
            pltpu.async_copy(table_hbm.at[idx_b[p]], rows_b[p], sg[p]).wait()
            if c + 2 < n_chunks:
                h_idx[p] = pltpu.async_copy(
                    idx_hbm.at[pl.ds(off(c + 2), chunk)], idx_b[p], si[p])
            h_s[p] = pltpu.async_copy(
                rows_b[p], out_hbm.at[pl.ds(off(c), chunk)], ss[p])
        h_s[0].wait()
        h_s[1].wait()

    return gather_kernel


def _build_gather_t(b, l, d, half):
    b_per_w = b // _NW
    nh = b_per_w // half
    assert nh == 2
    mesh = plsc.VectorSubcoreMesh(core_axis_name="c", subcore_axis_name="s")

    @functools.partial(
        pl.kernel,
        mesh=mesh,
        out_type=jax.ShapeDtypeStruct((l, d, b), jnp.float32),
        scratch_types=[
            pltpu.VMEM((half,), jnp.int32),
            pltpu.VMEM((half,), jnp.int32),
            pltpu.VMEM((half, d), jnp.float32),
            pltpu.VMEM((half, d), jnp.float32),
            pltpu.VMEM((d, half), jnp.float32),
            pltpu.VMEM((d, half), jnp.float32),
            pltpu.SemaphoreType.DMA,
            pltpu.SemaphoreType.DMA,
            pltpu.SemaphoreType.DMA,
            pltpu.SemaphoreType.DMA,
            pltpu.SemaphoreType.DMA,
            pltpu.SemaphoreType.DMA,
        ],
        compiler_params=pltpu.CompilerParams(
            use_tc_tiling_on_sc=False, needs_layout_passes=False),
    )
    def gather_t_kernel(table_hbm, xt_hbm, out_hbm,
                        idx0, idx1, rows0, rows1, tc0, tc1,
                        si0, si1, sg0, sg1, sw0, sw1):
        wid = lax.axis_index("s") * _NC + lax.axis_index("c")
        b0 = wid * b_per_w
        lane = lax.iota(jnp.int32, 16)
        rot = [(lane + dd) & 15 for dd in range(16)]

        def transpose_chunk(rows, tcols):
            def rbody(rt, _):
                rvec = rt * 16 + lane
                for jt in range(d // 16):
                    for g in range(0, 16, 8):
                        jv = [jt * 16 + rot[dd] for dd in range(g, g + 8)]
                        vals = [plsc.load_gather(rows, [rvec, jv[k]])
                                for k in range(8)]
                        for k in range(8):
                            plsc.store_scatter(tcols, [jv[k], rvec], vals[k])
                return ()
            lax.fori_loop(0, half // 16, rbody, (), unroll=1)

        def idx_src(li, h):
            return xt_hbm.at[li, pl.ds(b0 + h * half, half)]

        def out_dst(li, h):
            return out_hbm.at[li, :, pl.ds(b0 + h * half, half)]

        pltpu.async_copy(idx_src(0, 0), idx0, si0)
        pltpu.async_copy(idx_src(0, 1), idx1, si1)

        def body(i, _):
            pltpu.make_async_copy(idx_src(i, 0), idx0, si0).wait()
            pltpu.async_copy(table_hbm.at[idx0], rows0, sg0)
            pltpu.make_async_copy(idx_src(i, 1), idx1, si1).wait()
            pltpu.async_copy(table_hbm.at[idx1], rows1, sg1)

            pltpu.make_async_copy(table_hbm.at[idx0], rows0, sg0).wait()

            @pl.when(i < l - 1)
            def _():
                pltpu.async_copy(idx_src(i + 1, 0), idx0, si0)

            @pl.when(i > 0)
            def _():
                pltpu.make_async_copy(tc0, out_dst(i, 0), sw0).wait()
            transpose_chunk(rows0, tc0)
            pltpu.async_copy(tc0, out_dst(i, 0), sw0)

            pltpu.make_async_copy(table_hbm.at[idx1], rows1, sg1).wait()

            @pl.when(i < l - 1)
            def _():
                pltpu.async_copy(idx_src(i + 1, 1), idx1, si1)

            @pl.when(i > 0)
            def _():
                pltpu.make_async_copy(tc1, out_dst(i, 1), sw1).wait()
            transpose_chunk(rows1, tc1)
            pltpu.async_copy(tc1, out_dst(i, 1), sw1)
            return ()

        lax.fori_loop(0, l, body, (), unroll=False)
        pltpu.make_async_copy(tc0, out_dst(l - 1, 0), sw0).wait()
        pltpu.make_async_copy(tc1, out_dst(l - 1, 1), sw1).wait()

    return gather_t_kernel


def kernel(x, table):
    b, l = x.shape
    v, d = table.shape
    tt = table.T
    nfull = (v // 128) * 128
    tail_t = lax.slice(tt, (0, nfull), (d, v))
    tlin = _build_pack(v, d)(tt, tail_t)
    xt = x.T.astype(jnp.int32)
    out_t = _build_gather_t(b, l, d, 256)(tlin.reshape(v, d), xt)
    return out_t.transpose(2, 0, 1)


# speedup vs baseline: 3.2202x; 1.0226x over previous
import functools

import jax
import jax.numpy as jnp
from jax import lax
from jax.experimental import pallas as pl
from jax.experimental.pallas import tpu as pltpu
from jax.experimental.pallas import tpu_sc as plsc

_NW = 32
_NC = 2


def _build_pack(v, d):
    nblk = v // 128
    nmain = (nblk // _NW) * _NW
    slots = nmain // _NW
    nextra = nblk - nmain
    ntail = v - nblk * 128
    mesh = plsc.VectorSubcoreMesh(core_axis_name="c", subcore_axis_name="s")

    @functools.partial(
        pl.kernel,
        mesh=mesh,
        out_type=jax.ShapeDtypeStruct((v * d,), jnp.float32),
        scratch_types=[
            pltpu.VMEM((d, 128), jnp.float32),
            pltpu.VMEM((d, 128), jnp.float32),
            pltpu.VMEM((128 * d,), jnp.float32),
            pltpu.VMEM((128 * d,), jnp.float32),
            pltpu.VMEM((d, ntail), jnp.float32),
            pltpu.SemaphoreType.DMA,
            pltpu.SemaphoreType.DMA,
            pltpu.SemaphoreType.DMA,
            pltpu.SemaphoreType.DMA,
        ],
        compiler_params=pltpu.CompilerParams(
            use_tc_tiling_on_sc=True, needs_layout_passes=False),
    )
    def pack_kernel(tt_hbm, tail_hbm, out_hbm,
                    slab0, slab1, trows0, trows1, tailv, sr0, sr1, sw0, sw1):
        wid = lax.axis_index("s") * _NC + lax.axis_index("c")
        lane = lax.iota(jnp.int32, 16)

        def col_of(slot):
            return pl.multiple_of((slot * _NW + wid) * 128, 128)

        rot = [(lane + dd) & 15 for dd in range(16)]
        pos_pat = [lane * d + ((lane + dd) & 15) for dd in range(16)]

        def transpose_block(slab, trows):
            for jt in range(d // 16):
                j0 = jt * 16

                def cbody(ct, _):
                    c0 = ct * 16
                    cvec = c0 + lane
                    base = c0 * d + j0
                    for g in range(0, 16, 8):
                        vals = [plsc.load_gather(slab, [j0 + rot[dd], cvec])
                                for dd in range(g, g + 8)]
                        for k, dd in enumerate(range(g, g + 8)):
                            plsc.store_scatter(
                                trows, [base + pos_pat[dd]], vals[k])
                    return ()

                lax.fori_loop(0, 8, cbody, (), unroll=1)

        def rd(slot, slab, sem):
            return pltpu.async_copy(tt_hbm.at[:, pl.ds(col_of(slot), 128)],
                                    slab, sem)

        def wr(slot, trows, sem):
            off = pl.multiple_of(col_of(slot) * d, 8)
            return pltpu.async_copy(trows, out_hbm.at[pl.ds(off, 128 * d)], sem)

        rd(0, slab0, sr0)
        def body(i, _):
            slot_a = 2 * i
            slot_b = 2 * i + 1
            pltpu.make_async_copy(
                tt_hbm.at[:, pl.ds(col_of(slot_a), 128)], slab0, sr0).wait()
            rd(slot_b, slab1, sr1)

            @pl.when(i > 0)
            def _():
                pltpu.make_async_copy(
                    trows0, out_hbm.at[pl.ds(0, 128 * d)], sw0).wait()
            transpose_block(slab0, trows0)
            wr(slot_a, trows0, sw0)

            @pl.when(i < slots // 2 - 1)
            def _():
                rd(slot_a + 2, slab0, sr0)

            @pl.when(i > 0)
            def _():
                pltpu.make_async_copy(
                    trows1, out_hbm.at[pl.ds(0, 128 * d)], sw1).wait()
            pltpu.make_async_copy(
                tt_hbm.at[:, pl.ds(col_of(slot_b), 128)], slab1, sr1).wait()
            transpose_block(slab1, trows1)
            wr(slot_b, trows1, sw1)
            return ()

        lax.fori_loop(0, slots // 2, body, (), unroll=False)
        pltpu.make_async_copy(trows0, out_hbm.at[pl.ds(0, 128 * d)], sw0).wait()
        pltpu.make_async_copy(trows1, out_hbm.at[pl.ds(0, 128 * d)], sw1).wait()

        @pl.when(wid < nextra)
        def _():
            col0 = pl.multiple_of((nmain + wid) * 128, 128)
            pltpu.sync_copy(tt_hbm.at[:, pl.ds(col0, 128)], slab0)
            transpose_block(slab0, trows0)
            pltpu.sync_copy(trows0,
                            out_hbm.at[pl.ds(pl.multiple_of(col0 * d, 8),
                                             128 * d)])

        @pl.when(wid == _NW - 1)
        def _():
            pltpu.sync_copy(tail_hbm, tailv)
            def jbody(j, _):
                for m in range(ntail // 16):
                    vals = tailv[j, pl.ds(m * 16, 16)]
                    pos = (m * 16 + lane) * d + j
                    plsc.store_scatter(trows0, [pos], vals)
                return ()
            lax.fori_loop(0, d, jbody, (), unroll=False)
            pltpu.sync_copy(trows0.at[pl.ds(0, ntail * d)],
                            out_hbm.at[pl.ds(nblk * 128 * d, ntail * d)])

    return pack_kernel


def _build_gather(n, d, chunk):
    n_per_w = n // _NW
    n_chunks = n_per_w // chunk
    mesh = plsc.VectorSubcoreMesh(core_axis_name="c", subcore_axis_name="s")

    @functools.partial(
        pl.kernel,
        mesh=mesh,
        out_type=jax.ShapeDtypeStruct((n, d), jnp.float32),
        scratch_types=[
            pltpu.VMEM((chunk,), jnp.int32),
            pltpu.VMEM((chunk,), jnp.int32),
            pltpu.VMEM((chunk, d), jnp.float32),
            pltpu.VMEM((chunk, d), jnp.float32),
            pltpu.SemaphoreType.DMA,
            pltpu.SemaphoreType.DMA,
            pltpu.SemaphoreType.DMA,
            pltpu.SemaphoreType.DMA,
            pltpu.SemaphoreType.DMA,
            pltpu.SemaphoreType.DMA,
        ],
        compiler_params=pltpu.CompilerParams(use_tc_tiling_on_sc=False),
    )
    def gather_kernel(table_hbm, idx_hbm, out_hbm,
                      idx0, idx1, rows0, rows1, si0, si1, sg0, sg1, ss0, ss1):
        wid = lax.axis_index("s") * _NC + lax.axis_index("c")
        base = wid * n_per_w
        idx_b = [idx0, idx1]
        rows_b = [rows0, rows1]
        si = [si0, si1]
        sg = [sg0, sg1]
        ss = [ss0, ss1]

        def off(c):
            return base + c * chunk

        h_idx = [
            pltpu.async_copy(idx_hbm.at[pl.ds(off(0), chunk)], idx_b[0], si[0]),
            pltpu.async_copy(idx_hbm.at[pl.ds(off(1), chunk)], idx_b[1], si[1]),
        ]
        h_s = [None, None]
        for c in range(n_chunks):
            p = c % 2
            if c >= 2:
                h_s[p].wait()
            h_idx[p].wait()
            pltpu.async_copy(table_hbm.at[idx_b[p]], rows_b[p], sg[p]).wait()
            if c + 2 < n_chunks:
                h_idx[p] = pltpu.async_copy(
                    idx_hbm.at[pl.ds(off(c + 2), chunk)], idx_b[p], si[p])
            h_s[p] = pltpu.async_copy(
                rows_b[p], out_hbm.at[pl.ds(off(c), chunk)], ss[p])
        h_s[0].wait()
        h_s[1].wait()

    return gather_kernel


def _build_gather_t(b, l, d, half):
    b_per_w = b // _NW
    nh = b_per_w // half
    assert nh == 2
    mesh = plsc.VectorSubcoreMesh(core_axis_name="c", subcore_axis_name="s")

    @functools.partial(
        pl.kernel,
        mesh=mesh,
        out_type=jax.ShapeDtypeStruct((l, d, b), jnp.float32),
        scratch_types=[
            pltpu.VMEM((half,), jnp.int32),
            pltpu.VMEM((half,), jnp.int32),
            pltpu.VMEM((half, d), jnp.float32),
            pltpu.VMEM((half, d), jnp.float32),
            pltpu.VMEM((d, half), jnp.float32),
            pltpu.VMEM((d, half), jnp.float32),
            pltpu.SemaphoreType.DMA,
            pltpu.SemaphoreType.DMA,
            pltpu.SemaphoreType.DMA,
            pltpu.SemaphoreType.DMA,
            pltpu.SemaphoreType.DMA,
            pltpu.SemaphoreType.DMA,
        ],
        compiler_params=pltpu.CompilerParams(
            use_tc_tiling_on_sc=False, needs_layout_passes=False),
    )
    def gather_t_kernel(table_hbm, xt_hbm, out_hbm,
                        idx0, idx1, rows0, rows1, tc0, tc1,
                        si0, si1, sg0, sg1, sw0, sw1):
        wid = lax.axis_index("s") * _NC + lax.axis_index("c")
        b0 = wid * b_per_w
        lane = lax.iota(jnp.int32, 16)
        rot = [(lane + dd) & 15 for dd in range(16)]

        def transpose_chunk(rows, tcols):
            def rbody(rt, _):
                rvec = rt * 16 + lane
                for jt in range(d // 16):
                    for g in range(0, 16, 8):
                        jv = [jt * 16 + rot[dd] for dd in range(g, g + 8)]
                        vals = [plsc.load_gather(rows, [rvec, jv[k]])
                                for k in range(8)]
                        for k in range(8):
                            plsc.store_scatter(tcols, [jv[k], rvec], vals[k])
                return ()
            lax.fori_loop(0, half // 16, rbody, (), unroll=2)

        def idx_src(li, h):
            return xt_hbm.at[li, pl.ds(b0 + h * half, half)]

        def out_dst(li, h):
            return out_hbm.at[li, :, pl.ds(b0 + h * half, half)]

        pltpu.async_copy(idx_src(0, 0), idx0, si0)
        pltpu.async_copy(idx_src(0, 1), idx1, si1)

        def body(i, _):
            pltpu.make_async_copy(idx_src(i, 0), idx0, si0).wait()
            pltpu.async_copy(table_hbm.at[idx0], rows0, sg0)
            pltpu.make_async_copy(idx_src(i, 1), idx1, si1).wait()
            pltpu.async_copy(table_hbm.at[idx1], rows1, sg1)

            pltpu.make_async_copy(table_hbm.at[idx0], rows0, sg0).wait()

            @pl.when(i < l - 1)
            def _():
                pltpu.async_copy(idx_src(i + 1, 0), idx0, si0)

            @pl.when(i > 0)
            def _():
                pltpu.make_async_copy(tc0, out_dst(i, 0), sw0).wait()
            transpose_chunk(rows0, tc0)
            pltpu.async_copy(tc0, out_dst(i, 0), sw0)

            pltpu.make_async_copy(table_hbm.at[idx1], rows1, sg1).wait()

            @pl.when(i < l - 1)
            def _():
                pltpu.async_copy(idx_src(i + 1, 1), idx1, si1)

            @pl.when(i > 0)
            def _():
                pltpu.make_async_copy(tc1, out_dst(i, 1), sw1).wait()
            transpose_chunk(rows1, tc1)
            pltpu.async_copy(tc1, out_dst(i, 1), sw1)
            return ()

        lax.fori_loop(0, l, body, (), unroll=False)
        pltpu.make_async_copy(tc0, out_dst(l - 1, 0), sw0).wait()
        pltpu.make_async_copy(tc1, out_dst(l - 1, 1), sw1).wait()

    return gather_t_kernel


def kernel(x, table):
    b, l = x.shape
    v, d = table.shape
    tt = table.T
    nfull = (v // 128) * 128
    tail_t = lax.slice(tt, (0, nfull), (d, v))
    tlin = _build_pack(v, d)(tt, tail_t)
    xt = x.T.astype(jnp.int32)
    out_t = _build_gather_t(b, l, d, 256)(tlin.reshape(v, d), xt)
    return out_t.transpose(2, 0, 1)


# speedup vs baseline: 3.5958x; 1.1167x over previous
import functools

import jax
import jax.numpy as jnp
from jax import lax
from jax.experimental import pallas as pl
from jax.experimental.pallas import tpu as pltpu
from jax.experimental.pallas import tpu_sc as plsc

_NW = 32
_NC = 2


def _build_pack(v, d, w=256):
    nblk = v // w
    nmain = (nblk // _NW) * _NW
    slots = nmain // _NW
    nextra = nblk - nmain
    ntail = v - nblk * w
    mesh = plsc.VectorSubcoreMesh(core_axis_name="c", subcore_axis_name="s")

    @functools.partial(
        pl.kernel,
        mesh=mesh,
        out_type=jax.ShapeDtypeStruct((v * d,), jnp.float32),
        scratch_types=[
            pltpu.VMEM((d, w), jnp.float32),
            pltpu.VMEM((d, w), jnp.float32),
            pltpu.VMEM((w * d,), jnp.float32),
            pltpu.VMEM((w * d,), jnp.float32),
            pltpu.VMEM((d, ntail), jnp.float32),
            pltpu.SemaphoreType.DMA,
            pltpu.SemaphoreType.DMA,
            pltpu.SemaphoreType.DMA,
            pltpu.SemaphoreType.DMA,
        ],
        compiler_params=pltpu.CompilerParams(
            use_tc_tiling_on_sc=True, needs_layout_passes=False),
    )
    def pack_kernel(tt_hbm, tail_hbm, out_hbm,
                    slab0, slab1, trows0, trows1, tailv, sr0, sr1, sw0, sw1):
        wid = lax.axis_index("s") * _NC + lax.axis_index("c")
        lane = lax.iota(jnp.int32, 16)

        def col_of(slot):
            return pl.multiple_of((slot * _NW + wid) * w, 128)

        rot = [(lane + dd) & 15 for dd in range(16)]
        pos_pat = [lane * d + ((lane + dd) & 15) for dd in range(16)]

        def transpose_block(slab, trows):
            for jt in range(d // 16):
                j0 = jt * 16

                def cbody(ct, _):
                    c0 = ct * 16
                    cvec = c0 + lane
                    base = c0 * d + j0
                    for g in range(0, 16, 8):
                        vals = [plsc.load_gather(slab, [j0 + rot[dd], cvec])
                                for dd in range(g, g + 8)]
                        for k, dd in enumerate(range(g, g + 8)):
                            plsc.store_scatter(
                                trows, [base + pos_pat[dd]], vals[k])
                    return ()

                lax.fori_loop(0, w // 16, cbody, (), unroll=1)

        def rd(slot, slab, sem):
            return pltpu.async_copy(tt_hbm.at[:, pl.ds(col_of(slot), w)],
                                    slab, sem)

        def wr(slot, trows, sem):
            off = pl.multiple_of(col_of(slot) * d, 8)
            return pltpu.async_copy(trows, out_hbm.at[pl.ds(off, w * d)], sem)

        rd(0, slab0, sr0)
        def body(i, _):
            slot_a = 2 * i
            slot_b = 2 * i + 1
            pltpu.make_async_copy(
                tt_hbm.at[:, pl.ds(col_of(slot_a), w)], slab0, sr0).wait()
            rd(slot_b, slab1, sr1)

            @pl.when(i > 0)
            def _():
                pltpu.make_async_copy(
                    trows0, out_hbm.at[pl.ds(0, w * d)], sw0).wait()
            transpose_block(slab0, trows0)
            wr(slot_a, trows0, sw0)

            @pl.when(i < slots // 2 - 1)
            def _():
                rd(slot_a + 2, slab0, sr0)

            @pl.when(i > 0)
            def _():
                pltpu.make_async_copy(
                    trows1, out_hbm.at[pl.ds(0, w * d)], sw1).wait()
            pltpu.make_async_copy(
                tt_hbm.at[:, pl.ds(col_of(slot_b), w)], slab1, sr1).wait()
            transpose_block(slab1, trows1)
            wr(slot_b, trows1, sw1)
            return ()

        lax.fori_loop(0, slots // 2, body, (), unroll=False)
        pltpu.make_async_copy(trows0, out_hbm.at[pl.ds(0, w * d)], sw0).wait()
        pltpu.make_async_copy(trows1, out_hbm.at[pl.ds(0, w * d)], sw1).wait()

        @pl.when(wid < nextra)
        def _():
            col0 = pl.multiple_of((nmain + wid) * w, 128)
            pltpu.sync_copy(tt_hbm.at[:, pl.ds(col0, w)], slab0)
            transpose_block(slab0, trows0)
            pltpu.sync_copy(trows0,
                            out_hbm.at[pl.ds(pl.multiple_of(col0 * d, 8),
                                             w * d)])

        @pl.when(wid == _NW - 1)
        def _():
            pltpu.sync_copy(tail_hbm, tailv)
            def jbody(j, _):
                for m in range(ntail // 16):
                    vals = tailv[j, pl.ds(m * 16, 16)]
                    pos = (m * 16 + lane) * d + j
                    plsc.store_scatter(trows0, [pos], vals)
                return ()
            lax.fori_loop(0, d, jbody, (), unroll=False)
            pltpu.sync_copy(trows0.at[pl.ds(0, ntail * d)],
                            out_hbm.at[pl.ds(nblk * w * d, ntail * d)])

    return pack_kernel


def _build_gather(n, d, chunk):
    n_per_w = n // _NW
    n_chunks = n_per_w // chunk
    mesh = plsc.VectorSubcoreMesh(core_axis_name="c", subcore_axis_name="s")

    @functools.partial(
        pl.kernel,
        mesh=mesh,
        out_type=jax.ShapeDtypeStruct((n, d), jnp.float32),
        scratch_types=[
            pltpu.VMEM((chunk,), jnp.int32),
            pltpu.VMEM((chunk,), jnp.int32),
            pltpu.VMEM((chunk, d), jnp.float32),
            pltpu.VMEM((chunk, d), jnp.float32),
            pltpu.SemaphoreType.DMA,
            pltpu.SemaphoreType.DMA,
            pltpu.SemaphoreType.DMA,
            pltpu.SemaphoreType.DMA,
            pltpu.SemaphoreType.DMA,
            pltpu.SemaphoreType.DMA,
        ],
        compiler_params=pltpu.CompilerParams(use_tc_tiling_on_sc=False),
    )
    def gather_kernel(table_hbm, idx_hbm, out_hbm,
                      idx0, idx1, rows0, rows1, si0, si1, sg0, sg1, ss0, ss1):
        wid = lax.axis_index("s") * _NC + lax.axis_index("c")
        base = wid * n_per_w
        idx_b = [idx0, idx1]
        rows_b = [rows0, rows1]
        si = [si0, si1]
        sg = [sg0, sg1]
        ss = [ss0, ss1]

        def off(c):
            return base + c * chunk

        h_idx = [
            pltpu.async_copy(idx_hbm.at[pl.ds(off(0), chunk)], idx_b[0], si[0]),
            pltpu.async_copy(idx_hbm.at[pl.ds(off(1), chunk)], idx_b[1], si[1]),
        ]
        h_s = [None, None]
        for c in range(n_chunks):
            p = c % 2
            if c >= 2:
                h_s[p].wait()
            h_idx[p].wait()
            pltpu.async_copy(table_hbm.at[idx_b[p]], rows_b[p], sg[p]).wait()
            if c + 2 < n_chunks:
                h_idx[p] = pltpu.async_copy(
                    idx_hbm.at[pl.ds(off(c + 2), chunk)], idx_b[p], si[p])
            h_s[p] = pltpu.async_copy(
                rows_b[p], out_hbm.at[pl.ds(off(c), chunk)], ss[p])
        h_s[0].wait()
        h_s[1].wait()

    return gather_kernel


def _build_gather_t(b, l, d, half):
    b_per_w = b // _NW
    nh = b_per_w // half
    assert nh == 2
    mesh = plsc.VectorSubcoreMesh(core_axis_name="c", subcore_axis_name="s")

    @functools.partial(
        pl.kernel,
        mesh=mesh,
        out_type=jax.ShapeDtypeStruct((l, d, b), jnp.float32),
        scratch_types=[
            pltpu.VMEM((half,), jnp.int32),
            pltpu.VMEM((half,), jnp.int32),
            pltpu.VMEM((half, d), jnp.float32),
            pltpu.VMEM((half, d), jnp.float32),
            pltpu.VMEM((d, half), jnp.float32),
            pltpu.VMEM((d, half), jnp.float32),
            pltpu.SemaphoreType.DMA,
            pltpu.SemaphoreType.DMA,
            pltpu.SemaphoreType.DMA,
            pltpu.SemaphoreType.DMA,
            pltpu.SemaphoreType.DMA,
            pltpu.SemaphoreType.DMA,
        ],
        compiler_params=pltpu.CompilerParams(
            use_tc_tiling_on_sc=False, needs_layout_passes=False),
    )
    def gather_t_kernel(table_hbm, xt_hbm, out_hbm,
                        idx0, idx1, rows0, rows1, tc0, tc1,
                        si0, si1, sg0, sg1, sw0, sw1):
        wid = lax.axis_index("s") * _NC + lax.axis_index("c")
        b0 = wid * b_per_w
        lane = lax.iota(jnp.int32, 16)
        rot = [(lane + dd) & 15 for dd in range(16)]

        def transpose_chunk(rows, tcols):
            def rbody(rt, _):
                rvec = rt * 16 + lane
                for jt in range(d // 16):
                    for g in range(0, 16, 8):
                        jv = [jt * 16 + rot[dd] for dd in range(g, g + 8)]
                        vals = [plsc.load_gather(rows, [rvec, jv[k]])
                                for k in range(8)]
                        for k in range(8):
                            plsc.store_scatter(tcols, [jv[k], rvec], vals[k])
                return ()
            lax.fori_loop(0, half // 16, rbody, (), unroll=2)

        def idx_src(li, h):
            return xt_hbm.at[li, pl.ds(b0 + h * half, half)]

        def out_dst(li, h):
            return out_hbm.at[li, :, pl.ds(b0 + h * half, half)]

        pltpu.async_copy(idx_src(0, 0), idx0, si0)
        pltpu.async_copy(idx_src(0, 1), idx1, si1)

        def body(i, _):
            pltpu.make_async_copy(idx_src(i, 0), idx0, si0).wait()
            pltpu.async_copy(table_hbm.at[idx0], rows0, sg0)
            pltpu.make_async_copy(idx_src(i, 1), idx1, si1).wait()
            pltpu.async_copy(table_hbm.at[idx1], rows1, sg1)

            pltpu.make_async_copy(table_hbm.at[idx0], rows0, sg0).wait()

            @pl.when(i < l - 1)
            def _():
                pltpu.async_copy(idx_src(i + 1, 0), idx0, si0)

            @pl.when(i > 0)
            def _():
                pltpu.make_async_copy(tc0, out_dst(i, 0), sw0).wait()
            transpose_chunk(rows0, tc0)
            pltpu.async_copy(tc0, out_dst(i, 0), sw0)

            pltpu.make_async_copy(table_hbm.at[idx1], rows1, sg1).wait()

            @pl.when(i < l - 1)
            def _():
                pltpu.async_copy(idx_src(i + 1, 1), idx1, si1)

            @pl.when(i > 0)
            def _():
                pltpu.make_async_copy(tc1, out_dst(i, 1), sw1).wait()
            transpose_chunk(rows1, tc1)
            pltpu.async_copy(tc1, out_dst(i, 1), sw1)
            return ()

        lax.fori_loop(0, l, body, (), unroll=False)
        pltpu.make_async_copy(tc0, out_dst(l - 1, 0), sw0).wait()
        pltpu.make_async_copy(tc1, out_dst(l - 1, 1), sw1).wait()

    return gather_t_kernel


def kernel(x, table):
    b, l = x.shape
    v, d = table.shape
    tt = table.T
    nfull = (v // 128) * 128
    tail_t = lax.slice(tt, (0, nfull), (d, v))
    tlin = _build_pack(v, d)(tt, tail_t)
    xt = x.T.astype(jnp.int32)
    out_t = _build_gather_t(b, l, d, 256)(tlin.reshape(v, d), xt)
    return out_t.transpose(2, 0, 1)


# speedup vs baseline: 3.6040x; 1.0023x over previous
import functools

import jax
import jax.numpy as jnp
from jax import lax
from jax.experimental import pallas as pl
from jax.experimental.pallas import tpu as pltpu
from jax.experimental.pallas import tpu_sc as plsc

_NW = 32
_NC = 2


def _build_pack(v, d, w=256):
    nblk = v // w
    nmain = (nblk // _NW) * _NW
    slots = nmain // _NW
    nextra = nblk - nmain
    ntail = v - nblk * w
    mesh = plsc.VectorSubcoreMesh(core_axis_name="c", subcore_axis_name="s")

    @functools.partial(
        pl.kernel,
        mesh=mesh,
        out_type=jax.ShapeDtypeStruct((v * d,), jnp.float32),
        scratch_types=[
            pltpu.VMEM((d, w), jnp.float32),
            pltpu.VMEM((d, w), jnp.float32),
            pltpu.VMEM((w * d,), jnp.float32),
            pltpu.VMEM((w * d,), jnp.float32),
            pltpu.VMEM((d, ntail), jnp.float32),
            pltpu.SemaphoreType.DMA,
            pltpu.SemaphoreType.DMA,
            pltpu.SemaphoreType.DMA,
            pltpu.SemaphoreType.DMA,
        ],
        compiler_params=pltpu.CompilerParams(
            use_tc_tiling_on_sc=True, needs_layout_passes=False),
    )
    def pack_kernel(tt_hbm, tail_hbm, out_hbm,
                    slab0, slab1, trows0, trows1, tailv, sr0, sr1, sw0, sw1):
        wid = lax.axis_index("s") * _NC + lax.axis_index("c")
        lane = lax.iota(jnp.int32, 16)

        def col_of(slot):
            return pl.multiple_of((slot * _NW + wid) * w, 128)

        rot = [(lane + dd) & 15 for dd in range(16)]
        pos_pat = [lane * d + ((lane + dd) & 15) for dd in range(16)]

        def transpose_block(slab, trows):
            for jt in range(d // 16):
                j0 = jt * 16

                def cbody(ct, _):
                    c0 = ct * 16
                    cvec = c0 + lane
                    base = c0 * d + j0
                    for g in range(0, 16, 8):
                        vals = [plsc.load_gather(slab, [j0 + rot[dd], cvec])
                                for dd in range(g, g + 8)]
                        for k, dd in enumerate(range(g, g + 8)):
                            plsc.store_scatter(
                                trows, [base + pos_pat[dd]], vals[k])
                    return ()

                lax.fori_loop(0, w // 16, cbody, (), unroll=1)

        def rd(slot, slab, sem):
            return pltpu.async_copy(tt_hbm.at[:, pl.ds(col_of(slot), w)],
                                    slab, sem)

        def wr(slot, trows, sem):
            off = pl.multiple_of(col_of(slot) * d, 8)
            return pltpu.async_copy(trows, out_hbm.at[pl.ds(off, w * d)], sem)

        rd(0, slab0, sr0)
        def body(i, _):
            slot_a = 2 * i
            slot_b = 2 * i + 1
            pltpu.make_async_copy(
                tt_hbm.at[:, pl.ds(col_of(slot_a), w)], slab0, sr0).wait()
            rd(slot_b, slab1, sr1)

            @pl.when(i > 0)
            def _():
                pltpu.make_async_copy(
                    trows0, out_hbm.at[pl.ds(0, w * d)], sw0).wait()
            transpose_block(slab0, trows0)
            wr(slot_a, trows0, sw0)

            @pl.when(i < slots // 2 - 1)
            def _():
                rd(slot_a + 2, slab0, sr0)

            @pl.when(i > 0)
            def _():
                pltpu.make_async_copy(
                    trows1, out_hbm.at[pl.ds(0, w * d)], sw1).wait()
            pltpu.make_async_copy(
                tt_hbm.at[:, pl.ds(col_of(slot_b), w)], slab1, sr1).wait()
            transpose_block(slab1, trows1)
            wr(slot_b, trows1, sw1)
            return ()

        lax.fori_loop(0, slots // 2, body, (), unroll=False)
        pltpu.make_async_copy(trows0, out_hbm.at[pl.ds(0, w * d)], sw0).wait()
        pltpu.make_async_copy(trows1, out_hbm.at[pl.ds(0, w * d)], sw1).wait()

        @pl.when(wid < nextra)
        def _():
            col0 = pl.multiple_of((nmain + wid) * w, 128)
            pltpu.sync_copy(tt_hbm.at[:, pl.ds(col0, w)], slab0)
            transpose_block(slab0, trows0)
            pltpu.sync_copy(trows0,
                            out_hbm.at[pl.ds(pl.multiple_of(col0 * d, 8),
                                             w * d)])

        @pl.when(wid == _NW - 1)
        def _():
            pltpu.sync_copy(tail_hbm, tailv)
            def jbody(j, _):
                for m in range(ntail // 16):
                    vals = tailv[j, pl.ds(m * 16, 16)]
                    pos = (m * 16 + lane) * d + j
                    plsc.store_scatter(trows0, [pos], vals)
                return ()
            lax.fori_loop(0, d, jbody, (), unroll=False)
            pltpu.sync_copy(trows0.at[pl.ds(0, ntail * d)],
                            out_hbm.at[pl.ds(nblk * w * d, ntail * d)])

    return pack_kernel


def _build_gather_t(b, l, d, half):
    b_per_w = b // _NW
    nh = b_per_w // half
    assert nh == 2
    mesh = plsc.VectorSubcoreMesh(core_axis_name="c", subcore_axis_name="s")

    @functools.partial(
        pl.kernel,
        mesh=mesh,
        out_type=jax.ShapeDtypeStruct((l, d, b), jnp.float32),
        scratch_types=[
            pltpu.VMEM((half,), jnp.int32),
            pltpu.VMEM((half,), jnp.int32),
            pltpu.VMEM((half, d), jnp.float32),
            pltpu.VMEM((half, d), jnp.float32),
            pltpu.VMEM((d, half), jnp.float32),
            pltpu.VMEM((d, half), jnp.float32),
            pltpu.SemaphoreType.DMA,
            pltpu.SemaphoreType.DMA,
            pltpu.SemaphoreType.DMA,
            pltpu.SemaphoreType.DMA,
            pltpu.SemaphoreType.DMA,
            pltpu.SemaphoreType.DMA,
        ],
        compiler_params=pltpu.CompilerParams(
            use_tc_tiling_on_sc=False, needs_layout_passes=False),
    )
    def gather_t_kernel(table_hbm, xt_hbm, out_hbm,
                        idx0, idx1, rows0, rows1, tc0, tc1,
                        si0, si1, sg0, sg1, sw0, sw1):
        wid = lax.axis_index("s") * _NC + lax.axis_index("c")
        b0 = wid * b_per_w
        lane = lax.iota(jnp.int32, 16)
        rot = [(lane + dd) & 15 for dd in range(16)]

        def transpose_chunk(rows, tcols):
            def rbody(rt, _):
                rvec = rt * 16 + lane
                for jt in range(d // 16):
                    for g in range(0, 16, 8):
                        jv = [jt * 16 + rot[dd] for dd in range(g, g + 8)]
                        vals = [plsc.load_gather(rows, [rvec, jv[k]])
                                for k in range(8)]
                        for k in range(8):
                            plsc.store_scatter(tcols, [jv[k], rvec], vals[k])
                return ()
            lax.fori_loop(0, half // 16, rbody, (), unroll=2)

        def idx_src(li, h):
            return xt_hbm.at[li, pl.ds(b0 + h * half, half)]

        def out_dst(li, h):
            return out_hbm.at[li, :, pl.ds(b0 + h * half, half)]

        pltpu.async_copy(idx_src(0, 0), idx0, si0)
        pltpu.async_copy(idx_src(0, 1), idx1, si1)

        def body(i, _):
            pltpu.make_async_copy(idx_src(i, 0), idx0, si0).wait()
            pltpu.async_copy(table_hbm.at[idx0], rows0, sg0)
            pltpu.make_async_copy(idx_src(i, 1), idx1, si1).wait()
            pltpu.async_copy(table_hbm.at[idx1], rows1, sg1)

            pltpu.make_async_copy(table_hbm.at[idx0], rows0, sg0).wait()

            @pl.when(i < l - 1)
            def _():
                pltpu.async_copy(idx_src(i + 1, 0), idx0, si0)

            @pl.when(i > 0)
            def _():
                pltpu.make_async_copy(tc0, out_dst(i, 0), sw0).wait()
            transpose_chunk(rows0, tc0)
            pltpu.async_copy(tc0, out_dst(i, 0), sw0)

            pltpu.make_async_copy(table_hbm.at[idx1], rows1, sg1).wait()

            @pl.when(i < l - 1)
            def _():
                pltpu.async_copy(idx_src(i + 1, 1), idx1, si1)

            @pl.when(i > 0)
            def _():
                pltpu.make_async_copy(tc1, out_dst(i, 1), sw1).wait()
            transpose_chunk(rows1, tc1)
            pltpu.async_copy(tc1, out_dst(i, 1), sw1)
            return ()

        lax.fori_loop(0, l, body, (), unroll=False)
        pltpu.make_async_copy(tc0, out_dst(l - 1, 0), sw0).wait()
        pltpu.make_async_copy(tc1, out_dst(l - 1, 1), sw1).wait()

    return gather_t_kernel


def kernel(x, table):
    b, l = x.shape
    v, d = table.shape
    w = 256
    tt = table.T
    nfull = (v // w) * w
    tail_t = lax.slice(tt, (0, nfull), (d, v))
    tlin = _build_pack(v, d, w)(tt, tail_t)
    xt = x.T.astype(jnp.int32)
    out_t = _build_gather_t(b, l, d, 256)(tlin.reshape(v, d), xt)
    return out_t.transpose(2, 0, 1)
